# baseline (speedup 1.0000x reference)
import functools

import jax
import jax.numpy as jnp
from jax import lax
from jax.experimental import pallas as pl
from jax.experimental.pallas import tpu as pltpu
from jax.experimental.pallas import tpu_sc as plsc

N_ROWS = 16384
N_COLS = 256
K_OUT = 128
LANES = 16

SC_ROWS = 4096
TC_ROWS = N_ROWS - SC_ROWS

NUM_WORKERS = 32
ROWS_PER_WORKER = SC_ROWS // NUM_WORKERS
BLK_ROWS = 64
NBLK = ROWS_PER_WORKER // BLK_ROWS
NBUF_IN = 2
NBUF_OUT = 2

TC_BLK = 1024
TC_NBLK = TC_ROWS // TC_BLK


def _build_sc_kernel():
    mesh = plsc.VectorSubcoreMesh(core_axis_name="c", subcore_axis_name="s")

    @functools.partial(
        pl.kernel,
        mesh=mesh,
        out_type=jax.ShapeDtypeStruct((SC_ROWS, K_OUT), jnp.float32),
        compiler_params=pltpu.CompilerParams(
            needs_layout_passes=False,
            use_tc_tiling_on_sc=True,
        ),
        scratch_types=[
            pltpu.VMEM((NBUF_IN, BLK_ROWS, N_COLS), jnp.float32),
            pltpu.VMEM((NBUF_OUT, BLK_ROWS, K_OUT), jnp.float32),
            pltpu.SemaphoreType.DMA((NBUF_IN,)),
            pltpu.SemaphoreType.DMA((NBUF_OUT,)),
        ],
    )
    def k(in_hbm, out_hbm, in_v, out_v, in_sem, out_sem):
        wid = lax.axis_index("s") * 2 + lax.axis_index("c")
        lane2 = 2 * lax.iota(jnp.int32, LANES)
        cols = [lane2 + (2 * LANES * t) for t in range(K_OUT // LANES)]

        def row0(b):
            return wid * ROWS_PER_WORKER + b * BLK_ROWS

        def start_in(b):
            return pltpu.async_copy(
                in_hbm.at[pl.ds(TC_ROWS + row0(b), BLK_ROWS), :],
                in_v.at[b % NBUF_IN],
                in_sem.at[b % NBUF_IN],
            )

        def start_out(b):
            return pltpu.async_copy(
                out_v.at[b % NBUF_OUT],
                out_hbm.at[pl.ds(row0(b), BLK_ROWS), :],
                out_sem.at[b % NBUF_OUT],
            )

        in_copies = {b: start_in(b) for b in range(min(NBUF_IN, NBLK))}
        out_copies = {}
        for b in range(NBLK):
            in_copies.pop(b).wait()
            if b >= NBUF_OUT:
                out_copies.pop(b - NBUF_OUT).wait()

            src = in_v.at[b % NBUF_IN]
            dst = out_v.at[b % NBUF_OUT]

            @plsc.parallel_loop(0, BLK_ROWS, unroll=4)
            def body(r):
                rows = jnp.full((LANES,), r, jnp.int32)
                for t in range(K_OUT // LANES):
                    v = plsc.load_gather(src, [rows, cols[t]])
                    dst[r, pl.ds(LANES * t, LANES)] = v

            out_copies[b] = start_out(b)
            if b + NBUF_IN < NBLK:
                in_copies[b + NBUF_IN] = start_in(b + NBUF_IN)
        for b in sorted(out_copies):
            out_copies.pop(b).wait()

    return k


def _tc_body(x_ref, o_ref):
    ii = lax.broadcasted_iota(jnp.int32, (N_COLS, K_OUT), 0)
    jj = lax.broadcasted_iota(jnp.int32, (N_COLS, K_OUT), 1)
    sel = (ii == 2 * jj).astype(jnp.float32)
    o_ref[...] = jax.lax.dot_general(
        x_ref[...], sel,
        dimension_numbers=(((1,), (0,)), ((), ())),
        precision=jax.lax.Precision.DEFAULT,
        preferred_element_type=jnp.float32,
    )


_TC_KERNEL = pl.pallas_call(
    _tc_body,
    grid=(TC_NBLK,),
    in_specs=[pl.BlockSpec((TC_BLK, N_COLS), lambda i: (i, 0))],
    out_specs=pl.BlockSpec((TC_BLK, K_OUT), lambda i: (i, 0)),
    out_shape=jax.ShapeDtypeStruct((TC_ROWS, K_OUT), jnp.float32),
)

_SC_KERNEL = _build_sc_kernel()


def kernel(inputs):
    sc_out = _SC_KERNEL(inputs)
    tc_out = _TC_KERNEL(inputs[:TC_ROWS])
    return jnp.concatenate([tc_out, sc_out], axis=0)


# speedup vs baseline: 1.3430x; 1.3430x over previous
import functools

import jax
import jax.numpy as jnp
from jax import lax
from jax.experimental import pallas as pl
from jax.experimental.pallas import tpu as pltpu
from jax.experimental.pallas import tpu_sc as plsc

N_ROWS = 16384
N_COLS = 256
K_OUT = 128
NUM_WORKERS = 32
ROWS_PER_WORKER = N_ROWS // NUM_WORKERS
BLK_ROWS = 128
NBLK = ROWS_PER_WORKER // BLK_ROWS
LANES = 16


def _build_sc_kernel():
    mesh = plsc.VectorSubcoreMesh(core_axis_name="c", subcore_axis_name="s")

    @functools.partial(
        pl.kernel,
        mesh=mesh,
        out_type=jax.ShapeDtypeStruct((N_ROWS, K_OUT), jnp.float32),
        compiler_params=pltpu.CompilerParams(
            needs_layout_passes=False,
            use_tc_tiling_on_sc=True,
        ),
        scratch_types=[
            pltpu.VMEM((2, BLK_ROWS, N_COLS), jnp.float32),
            pltpu.VMEM((2, BLK_ROWS, K_OUT), jnp.float32),
            pltpu.SemaphoreType.DMA((2,)),
            pltpu.SemaphoreType.DMA((2,)),
        ],
    )
    def k(in_hbm, out_hbm, in_v, out_v, in_sem, out_sem):
        wid = lax.axis_index("s") * 2 + lax.axis_index("c")
        lane2 = 2 * lax.iota(jnp.int32, LANES)
        cols = [lane2 + (2 * LANES * t) for t in range(K_OUT // LANES)]

        def row0(b):
            return wid * ROWS_PER_WORKER + b * BLK_ROWS

        def start_in(b):
            return pltpu.async_copy(
                in_hbm.at[pl.ds(row0(b), BLK_ROWS), :],
                in_v.at[b % 2],
                in_sem.at[b % 2],
            )

        def start_out(b):
            return pltpu.async_copy(
                out_v.at[b % 2],
                out_hbm.at[pl.ds(row0(b), BLK_ROWS), :],
                out_sem.at[b % 2],
            )

        in_copies = {0: start_in(0)}
        out_copies = {}
        for b in range(NBLK):
            if b + 1 < NBLK:
                in_copies[b + 1] = start_in(b + 1)
            in_copies.pop(b).wait()
            if b >= 2:
                out_copies.pop(b - 2).wait()

            src = in_v.at[b % 2]
            dst = out_v.at[b % 2]

            @plsc.parallel_loop(0, BLK_ROWS, unroll=8)
            def body(r):
                rows = jnp.full((LANES,), r, jnp.int32)
                for t in range(K_OUT // LANES):
                    v = plsc.load_gather(src, [rows, cols[t]])
                    dst[r, pl.ds(LANES * t, LANES)] = v

            out_copies[b] = start_out(b)
        for b in sorted(out_copies):
            out_copies.pop(b).wait()

    return k


_SC_KERNEL = _build_sc_kernel()


def kernel(inputs):
    return _SC_KERNEL(inputs)


# speedup vs baseline: 1.4430x; 1.0745x over previous
import functools

import jax
import jax.numpy as jnp
from jax import lax
from jax.experimental import pallas as pl
from jax.experimental.pallas import tpu as pltpu
from jax.experimental.pallas import tpu_sc as plsc

N_ROWS = 16384
N_COLS = 256
K_OUT = 128
NUM_WORKERS = 32
ROWS_PER_WORKER = N_ROWS // NUM_WORKERS
BLK_ROWS = 128
NBLK = ROWS_PER_WORKER // BLK_ROWS
LANES = 16


def _build_sc_kernel():
    mesh = plsc.VectorSubcoreMesh(core_axis_name="c", subcore_axis_name="s")

    @functools.partial(
        pl.kernel,
        mesh=mesh,
        out_type=jax.ShapeDtypeStruct((N_ROWS, K_OUT), jnp.float32),
        compiler_params=pltpu.CompilerParams(
            needs_layout_passes=False,
            use_tc_tiling_on_sc=True,
        ),
        scratch_types=[
            pltpu.VMEM((2, BLK_ROWS, N_COLS), jnp.float32),
            pltpu.VMEM((2, BLK_ROWS, K_OUT), jnp.float32),
            pltpu.SemaphoreType.DMA((2,)),
            pltpu.SemaphoreType.DMA((2,)),
        ],
    )
    def k(in_hbm, out_hbm, in_v, out_v, in_sem, out_sem):
        wid = lax.axis_index("s") * 2 + lax.axis_index("c")
        lane2 = 2 * lax.iota(jnp.int32, LANES)
        cols = [lane2 + (2 * LANES * t) for t in range(K_OUT // LANES)]

        def row0(b):
            return wid * ROWS_PER_WORKER + b * BLK_ROWS

        def start_in(b):
            return pltpu.async_copy(
                in_hbm.at[pl.ds(row0(b), BLK_ROWS), :],
                in_v.at[b % 2],
                in_sem.at[b % 2],
            )

        def start_out(b):
            return pltpu.async_copy(
                out_v.at[b % 2],
                out_hbm.at[pl.ds(row0(b), BLK_ROWS), :],
                out_sem.at[b % 2],
            )

        in_copies = {0: start_in(0)}
        out_copies = {}
        for b in range(NBLK):
            if b + 1 < NBLK:
                in_copies[b + 1] = start_in(b + 1)
            in_copies.pop(b).wait()
            if b >= 2:
                out_copies.pop(b - 2).wait()

            src = in_v.at[b % 2]
            dst = out_v.at[b % 2]

            @plsc.parallel_loop(0, BLK_ROWS, unroll=4)
            def body(r):
                rows = jnp.full((LANES,), r, jnp.int32)
                for t in range(K_OUT // LANES):
                    v = plsc.load_gather(src, [rows, cols[t]])
                    dst[r, pl.ds(LANES * t, LANES)] = v

            out_copies[b] = start_out(b)
        for b in sorted(out_copies):
            out_copies.pop(b).wait()

    return k


_SC_KERNEL = _build_sc_kernel()


def kernel(inputs):
    return _SC_KERNEL(inputs)


# speedup vs baseline: 1.4482x; 1.0036x over previous
import functools

import jax
import jax.numpy as jnp
from jax import lax
from jax.experimental import pallas as pl
from jax.experimental.pallas import tpu as pltpu
from jax.experimental.pallas import tpu_sc as plsc

N_ROWS = 16384
N_COLS = 256
K_OUT = 128
NUM_WORKERS = 32
ROWS_PER_WORKER = N_ROWS // NUM_WORKERS
BLK_ROWS = 128
NBLK = ROWS_PER_WORKER // BLK_ROWS
LANES = 16


def _build_sc_kernel():
    mesh = plsc.VectorSubcoreMesh(core_axis_name="c", subcore_axis_name="s")

    @functools.partial(
        pl.kernel,
        mesh=mesh,
        out_type=jax.ShapeDtypeStruct((N_ROWS, K_OUT), jnp.float32),
        compiler_params=pltpu.CompilerParams(
            needs_layout_passes=False,
            use_tc_tiling_on_sc=True,
        ),
        scratch_types=[
            pltpu.VMEM((2, BLK_ROWS, N_COLS), jnp.float32),
            pltpu.VMEM((2, BLK_ROWS, K_OUT), jnp.float32),
            pltpu.SemaphoreType.DMA((2,)),
            pltpu.SemaphoreType.DMA((2,)),
        ],
    )
    def k(in_hbm, out_hbm, in_v, out_v, in_sem, out_sem):
        wid = lax.axis_index("s") * 2 + lax.axis_index("c")
        lane2 = 2 * lax.iota(jnp.int32, LANES)
        cols = [lane2 + (2 * LANES * t) for t in range(K_OUT // LANES)]

        def row0(b):
            return wid * ROWS_PER_WORKER + b * BLK_ROWS

        def start_in(b):
            return pltpu.async_copy(
                in_hbm.at[pl.ds(row0(b), BLK_ROWS), :],
                in_v.at[b % 2],
                in_sem.at[b % 2],
            )

        def start_out(b):
            return pltpu.async_copy(
                out_v.at[b % 2],
                out_hbm.at[pl.ds(row0(b), BLK_ROWS), :],
                out_sem.at[b % 2],
            )

        in_copies = {0: start_in(0)}
        out_copies = {}
        for b in range(NBLK):
            if b + 1 < NBLK:
                in_copies[b + 1] = start_in(b + 1)
            in_copies.pop(b).wait()
            if b >= 2:
                out_copies.pop(b - 2).wait()

            src = in_v.at[b % 2]
            dst = out_v.at[b % 2]

            @plsc.parallel_loop(0, BLK_ROWS, unroll=2)
            def body(r):
                rows = jnp.full((LANES,), r, jnp.int32)
                for t in range(K_OUT // LANES):
                    v = plsc.load_gather(src, [rows, cols[t]])
                    dst[r, pl.ds(LANES * t, LANES)] = v

            out_copies[b] = start_out(b)
        for b in sorted(out_copies):
            out_copies.pop(b).wait()

    return k


_SC_KERNEL = _build_sc_kernel()


def kernel(inputs):
    return _SC_KERNEL(inputs)


# speedup vs baseline: 1.4899x; 1.0288x over previous
import functools

import jax
import jax.numpy as jnp
from jax import lax
from jax.experimental import pallas as pl
from jax.experimental.pallas import tpu as pltpu
from jax.experimental.pallas import tpu_sc as plsc

N_ROWS = 16384
N_COLS = 256
K_OUT = 128
NUM_WORKERS = 32
ROWS_PER_WORKER = N_ROWS // NUM_WORKERS
BLK_ROWS = 128
NBLK = ROWS_PER_WORKER // BLK_ROWS
LANES = 16


def _build_sc_kernel():
    mesh = plsc.VectorSubcoreMesh(core_axis_name="c", subcore_axis_name="s")

    @functools.partial(
        pl.kernel,
        mesh=mesh,
        out_type=jax.ShapeDtypeStruct((N_ROWS, K_OUT), jnp.float32),
        compiler_params=pltpu.CompilerParams(
            needs_layout_passes=False,
            use_tc_tiling_on_sc=True,
        ),
        scratch_types=[
            pltpu.VMEM((2, BLK_ROWS, N_COLS), jnp.float32),
            pltpu.VMEM((2, BLK_ROWS, K_OUT), jnp.float32),
            pltpu.SemaphoreType.DMA((2,)),
            pltpu.SemaphoreType.DMA((2,)),
        ],
    )
    def k(in_hbm, out_hbm, in_v, out_v, in_sem, out_sem):
        wid = lax.axis_index("s") * 2 + lax.axis_index("c")
        lane2 = 2 * lax.iota(jnp.int32, LANES)
        cols = [lane2 + (2 * LANES * t) for t in range(K_OUT // LANES)]

        def row0(b):
            return wid * ROWS_PER_WORKER + b * BLK_ROWS

        def in_copy(b):
            par = lax.rem(b, 2)
            return pltpu.make_async_copy(
                in_hbm.at[pl.ds(row0(b), BLK_ROWS), :],
                in_v.at[par],
                in_sem.at[par],
            )

        def out_copy(b):
            par = lax.rem(b, 2)
            return pltpu.make_async_copy(
                out_v.at[par],
                out_hbm.at[pl.ds(row0(b), BLK_ROWS), :],
                out_sem.at[par],
            )

        in_copy(0).start()

        def block(b, _):
            @pl.when(b + 1 < NBLK)
            def _():
                in_copy(b + 1).start()

            in_copy(b).wait()

            @pl.when(b >= 2)
            def _():
                out_copy(b - 2).wait()

            par = lax.rem(b, 2)
            src = in_v.at[par]
            dst = out_v.at[par]

            @plsc.parallel_loop(0, BLK_ROWS, unroll=2)
            def body(r):
                rows = jnp.full((LANES,), r, jnp.int32)
                for t in range(K_OUT // LANES):
                    v = plsc.load_gather(src, [rows, cols[t]])
                    dst[r, pl.ds(LANES * t, LANES)] = v

            out_copy(b).start()
            return 0

        lax.fori_loop(0, NBLK, block, 0)
        out_copy(NBLK - 2).wait()
        out_copy(NBLK - 1).wait()

    return k


_SC_KERNEL = _build_sc_kernel()


def kernel(inputs):
    return _SC_KERNEL(inputs)


# speedup vs baseline: 1.5002x; 1.0069x over previous
import functools

import jax
import jax.numpy as jnp
from jax import lax
from jax.experimental import pallas as pl
from jax.experimental.pallas import tpu as pltpu
from jax.experimental.pallas import tpu_sc as plsc

N_ROWS = 16384
N_COLS = 256
K_OUT = 128
NUM_WORKERS = 32
ROWS_PER_WORKER = N_ROWS // NUM_WORKERS
BLK_ROWS = 64
NBLK = ROWS_PER_WORKER // BLK_ROWS
LANES = 16


def _build_sc_kernel():
    mesh = plsc.VectorSubcoreMesh(core_axis_name="c", subcore_axis_name="s")

    @functools.partial(
        pl.kernel,
        mesh=mesh,
        out_type=jax.ShapeDtypeStruct((N_ROWS, K_OUT), jnp.float32),
        compiler_params=pltpu.CompilerParams(
            needs_layout_passes=False,
            use_tc_tiling_on_sc=True,
        ),
        scratch_types=[
            pltpu.VMEM((2, BLK_ROWS, N_COLS), jnp.float32),
            pltpu.VMEM((2, BLK_ROWS, K_OUT), jnp.float32),
            pltpu.SemaphoreType.DMA((2,)),
            pltpu.SemaphoreType.DMA((2,)),
        ],
    )
    def k(in_hbm, out_hbm, in_v, out_v, in_sem, out_sem):
        wid = lax.axis_index("s") * 2 + lax.axis_index("c")
        lane2 = 2 * lax.iota(jnp.int32, LANES)
        cols = [lane2 + (2 * LANES * t) for t in range(K_OUT // LANES)]

        def row0(b):
            return wid * ROWS_PER_WORKER + b * BLK_ROWS

        def in_copy(b):
            par = lax.rem(b, 2)
            return pltpu.make_async_copy(
                in_hbm.at[pl.ds(row0(b), BLK_ROWS), :],
                in_v.at[par],
                in_sem.at[par],
            )

        def out_copy(b):
            par = lax.rem(b, 2)
            return pltpu.make_async_copy(
                out_v.at[par],
                out_hbm.at[pl.ds(row0(b), BLK_ROWS), :],
                out_sem.at[par],
            )

        in_copy(0).start()

        def block(b, _):
            @pl.when(b + 1 < NBLK)
            def _():
                in_copy(b + 1).start()

            in_copy(b).wait()

            @pl.when(b >= 2)
            def _():
                out_copy(b - 2).wait()

            par = lax.rem(b, 2)
            src = in_v.at[par]
            dst = out_v.at[par]

            @plsc.parallel_loop(0, BLK_ROWS, unroll=2)
            def body(r):
                rows = jnp.full((LANES,), r, jnp.int32)
                for t in range(K_OUT // LANES):
                    v = plsc.load_gather(src, [rows, cols[t]])
                    dst[r, pl.ds(LANES * t, LANES)] = v

            out_copy(b).start()
            return 0

        lax.fori_loop(0, NBLK, block, 0)
        out_copy(NBLK - 2).wait()
        out_copy(NBLK - 1).wait()

    return k


_SC_KERNEL = _build_sc_kernel()


def kernel(inputs):
    return _SC_KERNEL(inputs)
